# baseline (speedup 1.0000x reference)
import functools

import jax
import jax.numpy as jnp
from jax import lax
from jax.experimental import pallas as pl
from jax.experimental.pallas import tpu as pltpu
from jax.experimental.pallas import tpu_sc as plsc

NN = 10000
EE = 640000
NC = 2
NS = 16
NW = NC * NS
EPW = EE // NW
CHUNK = 2000
NCH = EPW // CHUNK
U = 5
GRP = CHUNK // 16


def _make_edge_kernel(H, C):
    K = H * C
    R = K + H
    mesh = plsc.VectorSubcoreMesh(core_axis_name="c", subcore_axis_name="s")

    @functools.partial(
        pl.kernel,
        mesh=mesh,
        out_type=jax.ShapeDtypeStruct((NW, R * NN), jnp.float32),
        scratch_types=[
            pltpu.VMEM((K * NN,), jnp.float32),
            pltpu.VMEM((H * NN,), jnp.float32),
            pltpu.VMEM((K * 16,), jnp.float32),
            pltpu.VMEM((H * 16,), jnp.float32),
            pltpu.VMEM((R * NN,), jnp.float32),
            pltpu.VMEM((2 * CHUNK,), jnp.int32),
            pltpu.VMEM((2 * CHUNK,), jnp.int32),
            pltpu.SemaphoreType.DMA,
            pltpu.SemaphoreType.DMA,
        ],
        compiler_params=pltpu.CompilerParams(needs_layout_passes=False),
    )
    def edge_kernel(htab_h, edtab_h, asrc_h, mx_h, src_h, dst_h, out_h,
                    htab, edtab, asv, mxv, acc, sbuf, dbuf, sem0, sem1):
        wid = lax.axis_index("s") * NC + lax.axis_index("c")
        base = wid * EPW
        sems = (sem0, sem1)

        pltpu.sync_copy(htab_h, htab)
        pltpu.sync_copy(edtab_h, edtab)
        pltpu.sync_copy(asrc_h, asv)
        pltpu.sync_copy(mx_h, mxv)

        zero16 = jnp.zeros((16,), jnp.float32)

        def zbody(i, carry):
            for u in range(5):
                acc[pl.ds(i * 80 + u * 16, 16)] = zero16
            return carry

        lax.fori_loop(0, R * NN // 80, zbody, 0)

        asplat = [asv[pl.ds(r * 16, 16)] for r in range(K)]
        msplat = [mxv[pl.ds(h * 16, 16)] for h in range(H)]

        def start_edges(slot, ci):
            off = base + ci * CHUNK
            pltpu.make_async_copy(
                src_h.at[pl.ds(off, CHUNK)],
                sbuf.at[pl.ds(slot * CHUNK, CHUNK)], sems[slot]).start()
            pltpu.make_async_copy(
                dst_h.at[pl.ds(off, CHUNK)],
                dbuf.at[pl.ds(slot * CHUNK, CHUNK)], sems[slot]).start()

        def wait_edges(slot):
            pltpu.make_async_copy(
                src_h.at[pl.ds(0, CHUNK)],
                sbuf.at[pl.ds(slot * CHUNK, CHUNK)], sems[slot]).wait()
            pltpu.make_async_copy(
                dst_h.at[pl.ds(0, CHUNK)],
                dbuf.at[pl.ds(slot * CHUNK, CHUNK)], sems[slot]).wait()

        def process(slot):
            sboff = slot * CHUNK

            def gbody(g, carry):
                for u in range(U):
                    o = sboff + (g * U + u) * 16
                    sv = sbuf[pl.ds(o, 16)]
                    dv = dbuf[pl.ds(o, 16)]
                    hv = [
                        plsc.load_gather(htab, [sv + jnp.int32(r * NN)])
                        for r in range(K)
                    ]
                    edv = [
                        plsc.load_gather(edtab, [dv + jnp.int32(h * NN)])
                        for h in range(H)
                    ]
                    for h in range(H):
                        es = hv[h * C] * asplat[h * C]
                        for c in range(1, C):
                            es = es + hv[h * C + c] * asplat[h * C + c]
                        al = es + edv[h]
                        al = jnp.maximum(al, al * jnp.float32(0.2))
                        m = edv[h] + msplat[h]
                        m = jnp.maximum(m, m * jnp.float32(0.2))
                        ex = jnp.exp(al - m)
                        plsc.addupdate_scatter(
                            acc, [dv + jnp.int32((K + h) * NN)], ex)
                        for c in range(C):
                            plsc.addupdate_scatter(
                                acc, [dv + jnp.int32((h * C + c) * NN)],
                                ex * hv[h * C + c])
                return carry

            lax.fori_loop(0, GRP // U, gbody, 0)

        start_edges(0, 0)

        def pairbody(p, carry):
            c0 = p * 2
            wait_edges(0)
            start_edges(1, c0 + 1)
            process(0)
            wait_edges(1)

            @pl.when(c0 + 2 < NCH)
            def _():
                start_edges(0, c0 + 2)

            process(1)
            return carry

        lax.fori_loop(0, NCH // 2, pairbody, 0)

        pltpu.sync_copy(acc, out_h.at[wid])

    return edge_kernel


_edge_l1 = _make_edge_kernel(2, 2)
_edge_l2 = _make_edge_kernel(2, 1)
_edge_l3 = _make_edge_kernel(1, 1)


VR = 4
NVCH = 78


def _make_sc_gemv():
    mesh = plsc.VectorSubcoreMesh(core_axis_name="c", subcore_axis_name="s")

    @functools.partial(
        pl.kernel,
        mesh=mesh,
        out_type=jax.ShapeDtypeStruct((NW, NN), jnp.float32),
        scratch_types=[
            pltpu.VMEM((NN,), jnp.float32),
            pltpu.VMEM((2 * VR * NN,), jnp.float32),
            pltpu.VMEM((NN,), jnp.float32),
            pltpu.SemaphoreType.DMA,
            pltpu.SemaphoreType.DMA,
        ],
        compiler_params=pltpu.CompilerParams(needs_layout_passes=False),
    )
    def gemv_sc(x_h, w_h, out_h, xtab, wbuf, acc, sem0, sem1):
        wid = lax.axis_index("s") * NC + lax.axis_index("c")
        base = wid * 312 + jnp.minimum(wid, 16)
        sems = (sem0, sem1)

        def startw(slot, ci):
            pltpu.make_async_copy(
                w_h.at[pl.ds((base + ci * VR) * NN, VR * NN)],
                wbuf.at[pl.ds(slot * VR * NN, VR * NN)], sems[slot]).start()

        def waitw(slot):
            pltpu.make_async_copy(
                w_h.at[pl.ds(0, VR * NN)],
                wbuf.at[pl.ds(slot * VR * NN, VR * NN)], sems[slot]).wait()

        startw(0, 0)
        pltpu.sync_copy(x_h, xtab)

        zero16 = jnp.zeros((16,), jnp.float32)

        def zb(i, carry):
            for u in range(5):
                acc[pl.ds(i * 80 + u * 16, 16)] = zero16
            return carry

        lax.fori_loop(0, NN // 80, zb, 0)

        idx0 = jnp.zeros((16,), jnp.int32)

        def proc(slot, ci):
            k0 = base + ci * VR
            xs = [plsc.load_gather(xtab, [idx0 + (k0 + r)])
                  for r in range(VR)]

            def jb(j, carry):
                for u in range(5):
                    o = (j * 5 + u) * 16
                    a = acc[pl.ds(o, 16)]
                    for r in range(VR):
                        a = a + xs[r] * wbuf[pl.ds(slot * VR * NN + r * NN + o, 16)]
                    acc[pl.ds(o, 16)] = a
                return carry

            lax.fori_loop(0, NN // 80, jb, 0)

        def pb(p, carry):
            c0 = p * 2
            waitw(0)
            startw(1, c0 + 1)
            proc(0, c0)
            waitw(1)

            @pl.when(c0 + 2 < NVCH)
            def _():
                startw(0, c0 + 2)

            proc(1, c0 + 1)
            return carry

        lax.fori_loop(0, NVCH // 2, pb, 0)

        @pl.when(wid < 16)
        def _():
            pltpu.sync_copy(w_h.at[pl.ds((base + 312) * NN, NN)],
                            wbuf.at[pl.ds(0, NN)])
            xs = plsc.load_gather(xtab, [idx0 + (base + 312)])

            def jb(j, carry):
                for u in range(5):
                    o = (j * 5 + u) * 16
                    acc[pl.ds(o, 16)] = (
                        acc[pl.ds(o, 16)] + xs * wbuf[pl.ds(o, 16)])
                return carry

            lax.fori_loop(0, NN // 80, jb, 0)

        pltpu.sync_copy(acc, out_h.at[wid])

    return gemv_sc


_gemv_sc = _make_sc_gemv()


def _expand_rows(den, H, C):
    if C == 1:
        return den
    rows = []
    for h in range(H):
        for _ in range(C):
            rows.append(den[h:h + 1, :])
    return jnp.concatenate(rows, axis=0)


def _prep_body(xt_ref, wT_ref, aS_ref, aD_ref, ht_ref, ed_ref, mx_ref, H):
    x = xt_ref[...]
    h = jnp.dot(wT_ref[...], x, preferred_element_type=jnp.float32)
    es = jnp.dot(aS_ref[...], h, preferred_element_type=jnp.float32)
    ed = jnp.dot(aD_ref[...], h, preferred_element_type=jnp.float32)
    ht_ref[...] = h
    ed_ref[...] = ed
    mx = jnp.max(es, axis=1)
    mx_ref[...] = jnp.broadcast_to(mx[:, None], (H, 16))


def _prep(xt, wT, aS, aD, H, K):
    body = functools.partial(_prep_body, H=H)
    return pl.pallas_call(
        body,
        out_shape=[
            jax.ShapeDtypeStruct((K, NN), jnp.float32),
            jax.ShapeDtypeStruct((H, NN), jnp.float32),
            jax.ShapeDtypeStruct((H, 16), jnp.float32),
        ],
    )(xt, wT, aS, aD)


def _combine_prep_body(parts_ref, b_ref, wT_ref, aS_ref, aD_ref,
                       ht_ref, ed_ref, mx_ref, Hp, Cp, H):
    Kp = Hp * Cp
    p = jnp.sum(parts_ref[...], axis=0)
    num = p[:Kp, :]
    den = _expand_rows(p[Kp:, :], Hp, Cp)
    x = jnp.where(den > 0, num / den, jnp.float32(0.0)) + b_ref[...]
    x = jnp.maximum(x, jnp.float32(0.0))
    h = jnp.dot(wT_ref[...], x, preferred_element_type=jnp.float32)
    es = jnp.dot(aS_ref[...], h, preferred_element_type=jnp.float32)
    ed = jnp.dot(aD_ref[...], h, preferred_element_type=jnp.float32)
    ht_ref[...] = h
    ed_ref[...] = ed
    mx = jnp.max(es, axis=1)
    mx_ref[...] = jnp.broadcast_to(mx[:, None], (H, 16))


def _combine_prep(parts, b, wT, aS, aD, Hp, Cp, H, K):
    body = functools.partial(_combine_prep_body, Hp=Hp, Cp=Cp, H=H)
    return pl.pallas_call(
        body,
        out_shape=[
            jax.ShapeDtypeStruct((K, NN), jnp.float32),
            jax.ShapeDtypeStruct((H, NN), jnp.float32),
            jax.ShapeDtypeStruct((H, 16), jnp.float32),
        ],
    )(parts, b, wT, aS, aD)


def _final_combine_body(parts_ref, b_ref, x_ref):
    p = jnp.sum(parts_ref[...], axis=0)
    num = p[0:1, :]
    den = p[1:2, :]
    x_ref[...] = jnp.where(den > 0, num / den, jnp.float32(0.0)) + b_ref[...]


def _final_combine(parts, b):
    return pl.pallas_call(
        _final_combine_body,
        out_shape=jax.ShapeDtypeStruct((1, NN), jnp.float32),
    )(parts, b)


def _gemv_body(x_ref, w_ref, b_ref, o_ref, BK):
    k = pl.program_id(1)

    @pl.when(k == 0)
    def _():
        o_ref[...] = b_ref[...]

    xk = x_ref[...].reshape(1, BK)
    o_ref[...] += jnp.dot(xk, w_ref[...],
                          preferred_element_type=jnp.float32)


def _gemv(x, W, b, BK=2000, BM=2048):
    Kd, Md = W.shape
    nk = Kd // BK
    nm = pl.cdiv(Md, BM)
    return pl.pallas_call(
        functools.partial(_gemv_body, BK=BK),
        grid=(nm, nk),
        in_specs=[
            pl.BlockSpec((1, 1, BK), lambda m, k: (k, 0, 0)),
            pl.BlockSpec((BK, BM), lambda m, k: (k, m)),
            pl.BlockSpec((1, BM), lambda m, k: (0, m)),
        ],
        out_specs=pl.BlockSpec((1, BM), lambda m, k: (0, m)),
        out_shape=jax.ShapeDtypeStruct((1, Md), jnp.float32),
        compiler_params=pltpu.CompilerParams(
            dimension_semantics=("parallel", "arbitrary")),
    )(x.reshape(nk, 1, BK), W, b)


def _head_body(l1_ref, v1p_ref, bc1_ref, wc2_ref, bc2_ref, mask_ref,
               probs_ref, value_ref, act_ref):
    l1 = l1_ref[...]
    maskf = mask_ref[...]
    p_ = jnp.where(maskf > 0, jnp.tanh(l1), jnp.float32(-999999.0))
    pmax = jnp.max(p_)
    e = jnp.exp(p_ - pmax)
    s = jnp.sum(e)
    probs = e / s
    probs_ref[...] = probs
    v1 = jnp.sum(v1p_ref[...], axis=0)[None, :] + bc1_ref[...]
    value_ref[...] = (jnp.sum(v1 * wc2_ref[...], axis=1,
                              keepdims=True) + bc2_ref[...])
    pm = jnp.max(probs)
    iota = lax.broadcasted_iota(jnp.int32, (1, NN), 1)
    cand = jnp.where(probs == pm, iota, jnp.int32(NN))
    act_ref[...] = jnp.min(cand, axis=1, keepdims=True)


def _head(l1, v1p, bc1, wc2t, bc2, maskf):
    return pl.pallas_call(
        _head_body,
        out_shape=[
            jax.ShapeDtypeStruct((1, NN), jnp.float32),
            jax.ShapeDtypeStruct((1, 1), jnp.float32),
            jax.ShapeDtypeStruct((1, 1), jnp.int32),
        ],
    )(l1, v1p, bc1, wc2t, bc2, maskf)


def _attn_mat(a):
    H, C = a.shape
    return (jnp.eye(H, dtype=a.dtype)[:, :, None] * a[None, :, :]).reshape(
        H, H * C)


def kernel(data, edge_index, edge_attr, W1, asrc1, adst1, b1,
           W2, asrc2, adst2, b2, W3, asrc3, adst3, b3,
           Wl1a, bl1a, Wl1b, bl1b, Wc1, bc1, Wc2, bc2, action_mask):
    del edge_attr
    src = edge_index[0]
    dst = edge_index[1]

    ht1, ed1, mx1 = _prep(data.T, W1.T, _attn_mat(asrc1), _attn_mat(adst1),
                          H=2, K=4)
    as1 = jnp.broadcast_to(asrc1.reshape(4, 1), (4, 16))
    parts1 = _edge_l1(ht1.reshape(-1), ed1.reshape(-1), as1.reshape(-1),
                      mx1.reshape(-1), src, dst)

    ht2, ed2, mx2 = _combine_prep(
        parts1.reshape(NW, 6, NN), b1.reshape(4, 1), W2.T,
        _attn_mat(asrc2), _attn_mat(adst2), Hp=2, Cp=2, H=2, K=2)
    as2 = jnp.broadcast_to(asrc2.reshape(2, 1), (2, 16))
    parts2 = _edge_l2(ht2.reshape(-1), ed2.reshape(-1), as2.reshape(-1),
                      mx2.reshape(-1), src, dst)

    ht3, ed3, mx3 = _combine_prep(
        parts2.reshape(NW, 4, NN), b2.reshape(2, 1), W3.T,
        _attn_mat(asrc3), _attn_mat(adst3), Hp=2, Cp=1, H=1, K=1)
    as3 = jnp.broadcast_to(asrc3.reshape(1, 1), (1, 16))
    parts3 = _edge_l3(ht3.reshape(-1), ed3.reshape(-1), as3.reshape(-1),
                      mx3.reshape(-1), src, dst)

    x3 = _final_combine(parts3.reshape(NW, 2, NN), b3.reshape(1, 1))

    t1 = _gemv(x3, Wl1a, bl1a.reshape(1, -1))
    v1p = _gemv_sc(x3.reshape(NN), Wc1.reshape(NN * NN))
    l1 = _gemv(t1, Wl1b, bl1b.reshape(1, -1))

    maskf = action_mask.astype(jnp.float32)
    probs, value, act = _head(l1, v1p, bc1.reshape(1, NN),
                              Wc2.reshape(1, NN), bc2.reshape(1, 1), maskf)
    return probs, value, act.reshape(1)


# speedup vs baseline: 1.2107x; 1.2107x over previous
import functools

import jax
import jax.numpy as jnp
from jax import lax
from jax.experimental import pallas as pl
from jax.experimental.pallas import tpu as pltpu
from jax.experimental.pallas import tpu_sc as plsc

NN = 10000
EE = 640000
NC = 2
NS = 16
NW = NC * NS
EPW = EE // NW
CHUNK = 2000
NCH = EPW // CHUNK
U = 5
GRP = CHUNK // 16


def _make_edge_kernel(H, C):
    K = H * C
    R = K + H
    mesh = plsc.VectorSubcoreMesh(core_axis_name="c", subcore_axis_name="s")

    @functools.partial(
        pl.kernel,
        mesh=mesh,
        out_type=jax.ShapeDtypeStruct((NW, R * NN), jnp.float32),
        scratch_types=[
            pltpu.VMEM((K * NN,), jnp.float32),
            pltpu.VMEM((H * NN,), jnp.float32),
            pltpu.VMEM((K * 16,), jnp.float32),
            pltpu.VMEM((H * 16,), jnp.float32),
            pltpu.VMEM((R * NN,), jnp.float32),
            pltpu.VMEM((2 * CHUNK,), jnp.int32),
            pltpu.VMEM((2 * CHUNK,), jnp.int32),
            pltpu.SemaphoreType.DMA,
            pltpu.SemaphoreType.DMA,
        ],
        compiler_params=pltpu.CompilerParams(needs_layout_passes=False),
    )
    def edge_kernel(htab_h, edtab_h, asrc_h, mx_h, src_h, dst_h, out_h,
                    htab, edtab, asv, mxv, acc, sbuf, dbuf, sem0, sem1):
        wid = lax.axis_index("s") * NC + lax.axis_index("c")
        base = wid * EPW
        sems = (sem0, sem1)

        pltpu.sync_copy(htab_h, htab)
        pltpu.sync_copy(edtab_h, edtab)
        pltpu.sync_copy(asrc_h, asv)
        pltpu.sync_copy(mx_h, mxv)

        zero16 = jnp.zeros((16,), jnp.float32)

        def zbody(i, carry):
            for u in range(5):
                acc[pl.ds(i * 80 + u * 16, 16)] = zero16
            return carry

        lax.fori_loop(0, R * NN // 80, zbody, 0)

        asplat = [asv[pl.ds(r * 16, 16)] for r in range(K)]
        msplat = [mxv[pl.ds(h * 16, 16)] for h in range(H)]

        def start_edges(slot, ci):
            off = base + ci * CHUNK
            pltpu.make_async_copy(
                src_h.at[pl.ds(off, CHUNK)],
                sbuf.at[pl.ds(slot * CHUNK, CHUNK)], sems[slot]).start()
            pltpu.make_async_copy(
                dst_h.at[pl.ds(off, CHUNK)],
                dbuf.at[pl.ds(slot * CHUNK, CHUNK)], sems[slot]).start()

        def wait_edges(slot):
            pltpu.make_async_copy(
                src_h.at[pl.ds(0, CHUNK)],
                sbuf.at[pl.ds(slot * CHUNK, CHUNK)], sems[slot]).wait()
            pltpu.make_async_copy(
                dst_h.at[pl.ds(0, CHUNK)],
                dbuf.at[pl.ds(slot * CHUNK, CHUNK)], sems[slot]).wait()

        def process(slot):
            sboff = slot * CHUNK

            def gbody(g, carry):
                for u in range(U):
                    o = sboff + (g * U + u) * 16
                    sv = sbuf[pl.ds(o, 16)]
                    dv = dbuf[pl.ds(o, 16)]
                    hv = [
                        plsc.load_gather(htab, [sv + jnp.int32(r * NN)])
                        for r in range(K)
                    ]
                    edv = [
                        plsc.load_gather(edtab, [dv + jnp.int32(h * NN)])
                        for h in range(H)
                    ]
                    for h in range(H):
                        es = hv[h * C] * asplat[h * C]
                        for c in range(1, C):
                            es = es + hv[h * C + c] * asplat[h * C + c]
                        al = es + edv[h]
                        al = jnp.maximum(al, al * jnp.float32(0.2))
                        m = edv[h] + msplat[h]
                        m = jnp.maximum(m, m * jnp.float32(0.2))
                        ex = jnp.exp(al - m)
                        plsc.addupdate_scatter(
                            acc, [dv + jnp.int32((K + h) * NN)], ex)
                        for c in range(C):
                            plsc.addupdate_scatter(
                                acc, [dv + jnp.int32((h * C + c) * NN)],
                                ex * hv[h * C + c])
                return carry

            lax.fori_loop(0, GRP // U, gbody, 0)

        start_edges(0, 0)

        def pairbody(p, carry):
            c0 = p * 2
            wait_edges(0)
            start_edges(1, c0 + 1)
            process(0)
            wait_edges(1)

            @pl.when(c0 + 2 < NCH)
            def _():
                start_edges(0, c0 + 2)

            process(1)
            return carry

        lax.fori_loop(0, NCH // 2, pairbody, 0)

        pltpu.sync_copy(acc, out_h.at[wid])

    return edge_kernel


_edge_l1 = _make_edge_kernel(2, 2)
_edge_l2 = _make_edge_kernel(2, 1)
_edge_l3 = _make_edge_kernel(1, 1)


def _expand_rows(den, H, C):
    if C == 1:
        return den
    rows = []
    for h in range(H):
        for _ in range(C):
            rows.append(den[h:h + 1, :])
    return jnp.concatenate(rows, axis=0)


def _prep_body(xt_ref, wT_ref, aS_ref, aD_ref, ht_ref, ed_ref, mx_ref, H):
    x = xt_ref[...]
    h = jnp.dot(wT_ref[...], x, preferred_element_type=jnp.float32)
    es = jnp.dot(aS_ref[...], h, preferred_element_type=jnp.float32)
    ed = jnp.dot(aD_ref[...], h, preferred_element_type=jnp.float32)
    ht_ref[...] = h
    ed_ref[...] = ed
    mx = jnp.max(es, axis=1)
    mx_ref[...] = jnp.broadcast_to(mx[:, None], (H, 16))


def _prep(xt, wT, aS, aD, H, K):
    body = functools.partial(_prep_body, H=H)
    return pl.pallas_call(
        body,
        out_shape=[
            jax.ShapeDtypeStruct((K, NN), jnp.float32),
            jax.ShapeDtypeStruct((H, NN), jnp.float32),
            jax.ShapeDtypeStruct((H, 16), jnp.float32),
        ],
    )(xt, wT, aS, aD)


def _combine_prep_body(parts_ref, b_ref, wT_ref, aS_ref, aD_ref,
                       ht_ref, ed_ref, mx_ref, Hp, Cp, H):
    Kp = Hp * Cp
    p = jnp.sum(parts_ref[...], axis=0)
    num = p[:Kp, :]
    den = _expand_rows(p[Kp:, :], Hp, Cp)
    x = jnp.where(den > 0, num / den, jnp.float32(0.0)) + b_ref[...]
    x = jnp.maximum(x, jnp.float32(0.0))
    h = jnp.dot(wT_ref[...], x, preferred_element_type=jnp.float32)
    es = jnp.dot(aS_ref[...], h, preferred_element_type=jnp.float32)
    ed = jnp.dot(aD_ref[...], h, preferred_element_type=jnp.float32)
    ht_ref[...] = h
    ed_ref[...] = ed
    mx = jnp.max(es, axis=1)
    mx_ref[...] = jnp.broadcast_to(mx[:, None], (H, 16))


def _combine_prep(parts, b, wT, aS, aD, Hp, Cp, H, K):
    body = functools.partial(_combine_prep_body, Hp=Hp, Cp=Cp, H=H)
    return pl.pallas_call(
        body,
        out_shape=[
            jax.ShapeDtypeStruct((K, NN), jnp.float32),
            jax.ShapeDtypeStruct((H, NN), jnp.float32),
            jax.ShapeDtypeStruct((H, 16), jnp.float32),
        ],
    )(parts, b, wT, aS, aD)


def _final_combine_body(parts_ref, b_ref, x_ref):
    p = jnp.sum(parts_ref[...], axis=0)
    num = p[0:1, :]
    den = p[1:2, :]
    x_ref[...] = jnp.where(den > 0, num / den, jnp.float32(0.0)) + b_ref[...]


def _final_combine(parts, b):
    return pl.pallas_call(
        _final_combine_body,
        out_shape=jax.ShapeDtypeStruct((1, NN), jnp.float32),
    )(parts, b)


def _gemv_body(x_ref, w_ref, b_ref, o_ref, BK):
    k = pl.program_id(1)

    @pl.when(k == 0)
    def _():
        o_ref[...] = b_ref[...]

    xk = x_ref[...].reshape(1, BK)
    o_ref[...] += jnp.dot(xk, w_ref[...],
                          preferred_element_type=jnp.float32)


def _gemv(x, W, b, BK=2000, BM=2048):
    Kd, Md = W.shape
    nk = Kd // BK
    nm = pl.cdiv(Md, BM)
    return pl.pallas_call(
        functools.partial(_gemv_body, BK=BK),
        grid=(nm, nk),
        in_specs=[
            pl.BlockSpec((1, 1, BK), lambda m, k: (k, 0, 0)),
            pl.BlockSpec((BK, BM), lambda m, k: (k, m)),
            pl.BlockSpec((1, BM), lambda m, k: (0, m)),
        ],
        out_specs=pl.BlockSpec((1, BM), lambda m, k: (0, m)),
        out_shape=jax.ShapeDtypeStruct((1, Md), jnp.float32),
        compiler_params=pltpu.CompilerParams(
            dimension_semantics=("parallel", "arbitrary")),
    )(x.reshape(nk, 1, BK), W, b)


def _head_body(l1_ref, v1_ref, wc2_ref, bc2_ref, mask_ref,
               probs_ref, value_ref, act_ref):
    l1 = l1_ref[...]
    maskf = mask_ref[...]
    p_ = jnp.where(maskf > 0, jnp.tanh(l1), jnp.float32(-999999.0))
    pmax = jnp.max(p_)
    e = jnp.exp(p_ - pmax)
    s = jnp.sum(e)
    probs = e / s
    probs_ref[...] = probs
    value_ref[...] = (jnp.sum(v1_ref[...] * wc2_ref[...], axis=1,
                              keepdims=True) + bc2_ref[...])
    pm = jnp.max(probs)
    iota = lax.broadcasted_iota(jnp.int32, (1, NN), 1)
    cand = jnp.where(probs == pm, iota, jnp.int32(NN))
    act_ref[...] = jnp.min(cand, axis=1, keepdims=True)


def _head(l1, v1, wc2t, bc2, maskf):
    return pl.pallas_call(
        _head_body,
        out_shape=[
            jax.ShapeDtypeStruct((1, NN), jnp.float32),
            jax.ShapeDtypeStruct((1, 1), jnp.float32),
            jax.ShapeDtypeStruct((1, 1), jnp.int32),
        ],
    )(l1, v1, wc2t, bc2, maskf)


def _attn_mat(a):
    H, C = a.shape
    return (jnp.eye(H, dtype=a.dtype)[:, :, None] * a[None, :, :]).reshape(
        H, H * C)


def kernel(data, edge_index, edge_attr, W1, asrc1, adst1, b1,
           W2, asrc2, adst2, b2, W3, asrc3, adst3, b3,
           Wl1a, bl1a, Wl1b, bl1b, Wc1, bc1, Wc2, bc2, action_mask):
    del edge_attr
    src = edge_index[0]
    dst = edge_index[1]

    ht1, ed1, mx1 = _prep(data.T, W1.T, _attn_mat(asrc1), _attn_mat(adst1),
                          H=2, K=4)
    as1 = jnp.broadcast_to(asrc1.reshape(4, 1), (4, 16))
    parts1 = _edge_l1(ht1.reshape(-1), ed1.reshape(-1), as1.reshape(-1),
                      mx1.reshape(-1), src, dst)

    ht2, ed2, mx2 = _combine_prep(
        parts1.reshape(NW, 6, NN), b1.reshape(4, 1), W2.T,
        _attn_mat(asrc2), _attn_mat(adst2), Hp=2, Cp=2, H=2, K=2)
    as2 = jnp.broadcast_to(asrc2.reshape(2, 1), (2, 16))
    parts2 = _edge_l2(ht2.reshape(-1), ed2.reshape(-1), as2.reshape(-1),
                      mx2.reshape(-1), src, dst)

    ht3, ed3, mx3 = _combine_prep(
        parts2.reshape(NW, 4, NN), b2.reshape(2, 1), W3.T,
        _attn_mat(asrc3), _attn_mat(adst3), Hp=2, Cp=1, H=1, K=1)
    as3 = jnp.broadcast_to(asrc3.reshape(1, 1), (1, 16))
    parts3 = _edge_l3(ht3.reshape(-1), ed3.reshape(-1), as3.reshape(-1),
                      mx3.reshape(-1), src, dst)

    x3 = _final_combine(parts3.reshape(NW, 2, NN), b3.reshape(1, 1))

    t1 = _gemv(x3, Wl1a, bl1a.reshape(1, -1))
    l1 = _gemv(t1, Wl1b, bl1b.reshape(1, -1))
    v1 = _gemv(x3, Wc1, bc1.reshape(1, -1))

    maskf = action_mask.astype(jnp.float32)
    probs, value, act = _head(l1, v1, Wc2.reshape(1, NN), bc2.reshape(1, 1),
                              maskf)
    return probs, value, act.reshape(1)
